# Initial kernel scaffold; baseline (speedup 1.0000x reference)
import jax
import jax.numpy as jnp
from jax.experimental import pallas as pl


def kernel(input, adj, W_fc, b_fc, W_cls, b_cls):
    raise NotImplementedError("write your pallas kernel here")


# speedup vs baseline: 9.1502x; 9.1502x over previous
import functools

import jax
import jax.numpy as jnp
from jax.experimental import pallas as pl
from jax.experimental.pallas import tpu as pltpu

_TAU = 0.5
_NLAYER = 2


def _fc_kernel(x_ref, w_ref, b_ref, o_ref):
    o_ref[...] = jax.lax.dot_general(
        x_ref[...], w_ref[...], (((1,), (1,)), ((), ())),
        preferred_element_type=jnp.float32) + b_ref[...]


def _adjmm_kernel(a_ref, h_ref, o_ref):
    o_ref[...] = jax.lax.dot_general(
        a_ref[...], h_ref[...], (((1,), (0,)), ((), ())),
        preferred_element_type=jnp.float32)


def _adjmm_norm_kernel(a_ref, h_ref, o_ref, n_ref):
    z = jax.lax.dot_general(
        a_ref[...], h_ref[...], (((1,), (0,)), ((), ())),
        preferred_element_type=jnp.float32)
    o_ref[...] = z
    nrm = jnp.sqrt(jnp.sum(z * z, axis=1, keepdims=True))
    n_ref[...] = z / jnp.maximum(nrm, 1e-12)


def _contrast_kernel(nb_ref, nall_ref, a_ref, loss_ref, *, bm, bk):
    i = pl.program_id(0)
    g = jax.lax.dot_general(
        nb_ref[...], nall_ref[...], (((1,), (1,)), ((), ())),
        preferred_element_type=jnp.float32)
    s = jnp.exp(g * (1.0 / _TAU))
    row = i * bm + jax.lax.broadcasted_iota(jnp.int32, (bm, bk), 0)
    col = jax.lax.broadcasted_iota(jnp.int32, (bm, bk), 1)
    dmask = row == col
    amask = jnp.logical_or(a_ref[...] > 0.0, dmask)
    pos = jnp.sum(jnp.where(amask, s, 0.0), axis=1, keepdims=True)
    rs = jnp.sum(s, axis=1, keepdims=True)
    dg = jnp.sum(jnp.where(dmask, s, 0.0), axis=1, keepdims=True)

    @pl.when(i == 0)
    def _():
        loss_ref[0, 0] = 0.0

    denom = 2.0 * rs - dg - pos
    ct = -jnp.log(pos / denom)
    loss_ref[0, 0] += jnp.sum(ct)


def _head_kernel(z_ref, w_ref, b_ref, y_ref):
    logits = jax.lax.dot_general(
        z_ref[...], w_ref[...], (((1,), (1,)), ((), ())),
        preferred_element_type=jnp.float32) + b_ref[...]
    m = jnp.max(logits, axis=1, keepdims=True)
    e = jnp.exp(logits - m)
    y_ref[...] = e / jnp.sum(e, axis=1, keepdims=True)


def kernel(input, adj, W_fc, b_fc, W_cls, b_cls):
    n, nf = input.shape
    hid = W_fc.shape[0]
    ncls = W_cls.shape[0]
    f32 = jnp.float32
    b_fc2 = b_fc.reshape(1, hid)
    b_cls2 = b_cls.reshape(1, ncls)

    BF = 1000
    h0 = pl.pallas_call(
        _fc_kernel,
        grid=(n // BF,),
        in_specs=[pl.BlockSpec((BF, nf), lambda i: (i, 0)),
                  pl.BlockSpec((hid, nf), lambda i: (0, 0)),
                  pl.BlockSpec((1, hid), lambda i: (0, 0))],
        out_specs=pl.BlockSpec((BF, hid), lambda i: (i, 0)),
        out_shape=jax.ShapeDtypeStruct((n, hid), f32),
    )(input, W_fc, b_fc2)

    BM = 200
    h1 = pl.pallas_call(
        _adjmm_kernel,
        grid=(n // BM,),
        in_specs=[pl.BlockSpec((BM, n), lambda i: (i, 0)),
                  pl.BlockSpec((n, hid), lambda i: (0, 0))],
        out_specs=pl.BlockSpec((BM, hid), lambda i: (i, 0)),
        out_shape=jax.ShapeDtypeStruct((n, hid), f32),
    )(adj, h0)

    z, nz = pl.pallas_call(
        _adjmm_norm_kernel,
        grid=(n // BM,),
        in_specs=[pl.BlockSpec((BM, n), lambda i: (i, 0)),
                  pl.BlockSpec((n, hid), lambda i: (0, 0))],
        out_specs=[pl.BlockSpec((BM, hid), lambda i: (i, 0)),
                   pl.BlockSpec((BM, hid), lambda i: (i, 0))],
        out_shape=[jax.ShapeDtypeStruct((n, hid), f32),
                   jax.ShapeDtypeStruct((n, hid), f32)],
    )(adj, h1)

    BMc = 200
    loss_sum = pl.pallas_call(
        functools.partial(_contrast_kernel, bm=BMc, bk=n),
        grid=(n // BMc,),
        in_specs=[pl.BlockSpec((BMc, hid), lambda i: (i, 0)),
                  pl.BlockSpec((n, hid), lambda i: (0, 0)),
                  pl.BlockSpec((BMc, n), lambda i: (i, 0))],
        out_specs=pl.BlockSpec(memory_space=pltpu.SMEM),
        out_shape=jax.ShapeDtypeStruct((1, 1), f32),
    )(nz, nz, adj)

    y = pl.pallas_call(
        _head_kernel,
        grid=(n // BF,),
        in_specs=[pl.BlockSpec((BF, hid), lambda i: (i, 0)),
                  pl.BlockSpec((ncls, hid), lambda i: (0, 0)),
                  pl.BlockSpec((1, ncls), lambda i: (0, 0))],
        out_specs=pl.BlockSpec((BF, ncls), lambda i: (i, 0)),
        out_shape=jax.ShapeDtypeStruct((n, ncls), f32),
    )(z, W_cls, b_cls2)

    loss = (loss_sum[0, 0] * (_NLAYER / n)).astype(f32)
    return (y, loss)


# speedup vs baseline: 9.7987x; 1.0709x over previous
import functools

import jax
import jax.numpy as jnp
from jax.experimental import pallas as pl
from jax.experimental.pallas import tpu as pltpu

_TAU = 0.5
_NLAYER = 2


def _fc_kernel(x_ref, w_ref, b_ref, o_ref):
    o_ref[...] = jax.lax.dot_general(
        x_ref[...], w_ref[...], (((1,), (1,)), ((), ())),
        preferred_element_type=jnp.float32) + b_ref[...]


def _adjmm_kernel(a_ref, h_ref, o_ref):
    o_ref[...] = jax.lax.dot_general(
        a_ref[...], h_ref[...], (((1,), (0,)), ((), ())),
        preferred_element_type=jnp.float32)


def _adjmm_norm_kernel(a_ref, h_ref, o_ref, n_ref):
    z = jax.lax.dot_general(
        a_ref[...], h_ref[...], (((1,), (0,)), ((), ())),
        preferred_element_type=jnp.float32)
    o_ref[...] = z
    nrm = jnp.sqrt(jnp.sum(z * z, axis=1, keepdims=True))
    n_ref[...] = z / jnp.maximum(nrm, 1e-12)


def _contrast_kernel(nb_ref, nall_ref, a_ref, loss_ref, *, bm, bk):
    i = pl.program_id(0)
    g = jax.lax.dot_general(
        nb_ref[...], nall_ref[...], (((1,), (1,)), ((), ())),
        preferred_element_type=jnp.float32)
    s = jnp.exp(g * (1.0 / _TAU))
    row = i * bm + jax.lax.broadcasted_iota(jnp.int32, (bm, bk), 0)
    col = jax.lax.broadcasted_iota(jnp.int32, (bm, bk), 1)
    dmask = row == col
    amask = jnp.logical_or(a_ref[...] > 0.0, dmask)
    pos = jnp.sum(jnp.where(amask, s, 0.0), axis=1, keepdims=True)
    rs = jnp.sum(s, axis=1, keepdims=True)
    dg = jnp.sum(jnp.where(dmask, s, 0.0), axis=1, keepdims=True)

    @pl.when(i == 0)
    def _():
        loss_ref[0, 0] = 0.0

    denom = 2.0 * rs - dg - pos
    ct = -jnp.log(pos / denom)
    loss_ref[0, 0] += jnp.sum(ct)


def _head_kernel(z_ref, w_ref, b_ref, y_ref):
    logits = jax.lax.dot_general(
        z_ref[...], w_ref[...], (((1,), (1,)), ((), ())),
        preferred_element_type=jnp.float32) + b_ref[...]
    m = jnp.max(logits, axis=1, keepdims=True)
    e = jnp.exp(logits - m)
    y_ref[...] = e / jnp.sum(e, axis=1, keepdims=True)


def kernel(input, adj, W_fc, b_fc, W_cls, b_cls):
    n, nf = input.shape
    hid = W_fc.shape[0]
    ncls = W_cls.shape[0]
    f32 = jnp.float32
    b_fc2 = b_fc.reshape(1, hid)
    b_cls2 = b_cls.reshape(1, ncls)

    BF = 1000
    h0 = pl.pallas_call(
        _fc_kernel,
        grid=(n // BF,),
        in_specs=[pl.BlockSpec((BF, nf), lambda i: (i, 0)),
                  pl.BlockSpec((hid, nf), lambda i: (0, 0)),
                  pl.BlockSpec((1, hid), lambda i: (0, 0))],
        out_specs=pl.BlockSpec((BF, hid), lambda i: (i, 0)),
        out_shape=jax.ShapeDtypeStruct((n, hid), f32),
    )(input, W_fc, b_fc2)

    BM = 400
    h1 = pl.pallas_call(
        _adjmm_kernel,
        grid=(n // BM,),
        in_specs=[pl.BlockSpec((BM, n), lambda i: (i, 0)),
                  pl.BlockSpec((n, hid), lambda i: (0, 0))],
        out_specs=pl.BlockSpec((BM, hid), lambda i: (i, 0)),
        out_shape=jax.ShapeDtypeStruct((n, hid), f32),
    )(adj, h0)

    z, nz = pl.pallas_call(
        _adjmm_norm_kernel,
        grid=(n // BM,),
        in_specs=[pl.BlockSpec((BM, n), lambda i: (i, 0)),
                  pl.BlockSpec((n, hid), lambda i: (0, 0))],
        out_specs=[pl.BlockSpec((BM, hid), lambda i: (i, 0)),
                   pl.BlockSpec((BM, hid), lambda i: (i, 0))],
        out_shape=[jax.ShapeDtypeStruct((n, hid), f32),
                   jax.ShapeDtypeStruct((n, hid), f32)],
    )(adj, h1)

    BMc = 400
    loss_sum = pl.pallas_call(
        functools.partial(_contrast_kernel, bm=BMc, bk=n),
        grid=(n // BMc,),
        in_specs=[pl.BlockSpec((BMc, hid), lambda i: (i, 0)),
                  pl.BlockSpec((n, hid), lambda i: (0, 0)),
                  pl.BlockSpec((BMc, n), lambda i: (i, 0))],
        out_specs=pl.BlockSpec(memory_space=pltpu.SMEM),
        out_shape=jax.ShapeDtypeStruct((1, 1), f32),
    )(nz, nz, adj)

    y = pl.pallas_call(
        _head_kernel,
        grid=(n // BF,),
        in_specs=[pl.BlockSpec((BF, hid), lambda i: (i, 0)),
                  pl.BlockSpec((ncls, hid), lambda i: (0, 0)),
                  pl.BlockSpec((1, ncls), lambda i: (0, 0))],
        out_specs=pl.BlockSpec((BF, ncls), lambda i: (i, 0)),
        out_shape=jax.ShapeDtypeStruct((n, ncls), f32),
    )(z, W_cls, b_cls2)

    loss = (loss_sum[0, 0] * (_NLAYER / n)).astype(f32)
    return (y, loss)
